# baseline (speedup 1.0000x reference)
import jax
import jax.numpy as jnp
from jax.experimental import pallas as pl

_DEPTH = 1000
_ROWS = 16384
_BLOCK = 1024


def _one_hot_body(idx_ref, out_ref):
    idx = idx_ref[...]
    cols = jax.lax.broadcasted_iota(jnp.int32, (_BLOCK, _DEPTH), 1)
    out_ref[...] = jnp.where(idx == cols, jnp.float32(1.0), jnp.float32(0.0))


def kernel(inputs):
    idx = inputs.astype(jnp.int32)
    return pl.pallas_call(
        _one_hot_body,
        grid=(_ROWS // _BLOCK,),
        in_specs=[pl.BlockSpec((_BLOCK, 1), lambda i: (i, 0))],
        out_specs=pl.BlockSpec((_BLOCK, _DEPTH), lambda i: (i, 0)),
        out_shape=jax.ShapeDtypeStruct((_ROWS, _DEPTH), jnp.float32),
    )(idx)


# speedup vs baseline: 1.0233x; 1.0233x over previous
import jax
import jax.numpy as jnp
from jax.experimental import pallas as pl

_DEPTH = 1000
_ROWS = 16384
_BLOCK = 4096


def _one_hot_body(idx_ref, out_ref):
    idx = idx_ref[...]
    cols = jax.lax.broadcasted_iota(jnp.int32, (_BLOCK, _DEPTH), 1)
    out_ref[...] = jnp.where(idx == cols, jnp.float32(1.0), jnp.float32(0.0))


def kernel(inputs):
    idx = inputs.astype(jnp.int32)
    return pl.pallas_call(
        _one_hot_body,
        grid=(_ROWS // _BLOCK,),
        in_specs=[pl.BlockSpec((_BLOCK, 1), lambda i: (i, 0))],
        out_specs=pl.BlockSpec((_BLOCK, _DEPTH), lambda i: (i, 0)),
        out_shape=jax.ShapeDtypeStruct((_ROWS, _DEPTH), jnp.float32),
    )(idx)


# speedup vs baseline: 1.0295x; 1.0060x over previous
import jax
import jax.numpy as jnp
from jax.experimental import pallas as pl

_DEPTH = 1000
_ROWS = 16384
_BLOCK = 1024


def _one_hot_body(idx_ref, out_ref):
    i = pl.program_id(0)
    idx = idx_ref[pl.ds(i * _BLOCK, _BLOCK), :]
    cols = jax.lax.broadcasted_iota(jnp.int32, (_BLOCK, _DEPTH), 1)
    out_ref[...] = jnp.where(idx == cols, jnp.float32(1.0), jnp.float32(0.0))


def kernel(inputs):
    idx = inputs.astype(jnp.int32)
    return pl.pallas_call(
        _one_hot_body,
        grid=(_ROWS // _BLOCK,),
        in_specs=[pl.BlockSpec((_ROWS, 1), lambda i: (0, 0))],
        out_specs=pl.BlockSpec((_BLOCK, _DEPTH), lambda i: (i, 0)),
        out_shape=jax.ShapeDtypeStruct((_ROWS, _DEPTH), jnp.float32),
    )(idx)
